# baseline (speedup 1.0000x reference)
import functools

import jax
import jax.numpy as jnp
from jax import lax
from jax.experimental import pallas as pl
from jax.experimental.pallas import tpu as pltpu
from jax.experimental.pallas import tpu_sc as plsc

_N = 10000
_D = 128
_S = 2500
_R2 = 0.3 * 0.3
_K = 64
_NP = 10240
_SP = 2560
_PER = 80
_NG = _NP // 16


def _fps_body(px_ref, py_ref, pz_ref, sel_ref, pdx_ref, pdy_ref, pdz_ref):
    px = px_ref[:]
    py = py_ref[:]
    pz = pz_ref[:]
    r = lax.broadcasted_iota(jnp.int32, (80, 128), 0)
    c = lax.broadcasted_iota(jnp.int32, (80, 128), 1)
    lin = r * 128 + c
    validm = lin < _N
    zero = jnp.float32(0.0)

    eq0 = lin == 0
    px0 = jnp.sum(jnp.where(eq0, px, zero))
    py0 = jnp.sum(jnp.where(eq0, py, zero))
    pz0 = jnp.sum(jnp.where(eq0, pz, zero))
    dx = px - px0
    dy = py - py0
    dz = pz - pz0
    mind = dx * dx + dy * dy + dz * dz
    mind = jnp.where(validm, mind, -jnp.inf)
    sel_ref[0] = jnp.int32(0)
    pdx_ref[0] = px0
    pdy_ref[0] = py0
    pdz_ref[0] = pz0

    def pick(a, b):
        av, ai, ax, ay, az = a
        bv, bi, bx, by, bz = b
        t = (bv > av) | ((bv == av) & (bi < ai))
        return (jnp.where(t, bv, av), jnp.where(t, bi, ai),
                jnp.where(t, bx, ax), jnp.where(t, by, ay),
                jnp.where(t, bz, az))

    def body(i, mind):
        parts = [
            (mind[k * 8:(k + 1) * 8], lin[k * 8:(k + 1) * 8],
             px[k * 8:(k + 1) * 8], py[k * 8:(k + 1) * 8],
             pz[k * 8:(k + 1) * 8])
            for k in range(10)
        ]
        while len(parts) > 1:
            nxt_parts = [pick(parts[j], parts[j + 1])
                         for j in range(0, len(parts) - 1, 2)]
            if len(parts) % 2:
                nxt_parts.append(parts[-1])
            parts = nxt_parts
        w = parts[0]
        for sh in (4, 2, 1):
            rolled = tuple(
                jnp.concatenate([a[sh:], a[:sh]], axis=0) for a in w)
            w = pick(w, rolled)
        for sh in (64, 32, 16, 8, 4, 2, 1):
            rolled = tuple(
                jnp.concatenate([a[:, sh:], a[:, :sh]], axis=1) for a in w)
            w = pick(w, rolled)
        _, wi, wx, wy, wz = w
        ddx = px - wx[0:1, :]
        ddy = py - wy[0:1, :]
        ddz = pz - wz[0:1, :]
        d = ddx * ddx + ddy * ddy + ddz * ddz
        sel_ref[i] = wi[0, 0]
        pdx_ref[i] = wx[0, 0]
        pdy_ref[i] = wy[0, 0]
        pdz_ref[i] = wz[0, 0]
        return jnp.minimum(mind, d)

    lax.fori_loop(1, _S, body, mind)


def _fps(px2d, py2d, pz2d):
    return pl.pallas_call(
        _fps_body,
        out_shape=[
            jax.ShapeDtypeStruct((_S,), jnp.int32),
            jax.ShapeDtypeStruct((_S,), jnp.float32),
            jax.ShapeDtypeStruct((_S,), jnp.float32),
            jax.ShapeDtypeStruct((_S,), jnp.float32),
        ],
        out_specs=[
            pl.BlockSpec(memory_space=pltpu.SMEM),
            pl.BlockSpec(memory_space=pltpu.SMEM),
            pl.BlockSpec(memory_space=pltpu.SMEM),
            pl.BlockSpec(memory_space=pltpu.SMEM),
        ],
    )(px2d, py2d, pz2d)


def _u_body(x_ref, p_ref, wa_ref, wb_ref, o_ref):
    o_ref[:] = jnp.dot(
        x_ref[:], wa_ref[:], preferred_element_type=jnp.float32
    ) + jnp.dot(p_ref[:], wb_ref[:], preferred_element_type=jnp.float32)


def _u_matmul(xp, posp8, wa, wbp):
    return pl.pallas_call(
        _u_body,
        grid=(20,),
        in_specs=[
            pl.BlockSpec((512, _D), lambda i: (i, 0)),
            pl.BlockSpec((512, 8), lambda i: (i, 0)),
            pl.BlockSpec((_D, _D), lambda i: (0, 0)),
            pl.BlockSpec((8, _D), lambda i: (0, 0)),
        ],
        out_specs=pl.BlockSpec((512, _D), lambda i: (i, 0)),
        out_shape=jax.ShapeDtypeStruct((_NP, _D), jnp.float32),
    )(xp, posp8, wa, wbp)


def _hm_body(cx_ref, cy_ref, cz_ref, px_ref, py_ref, pz_ref, par_ref,
             pk_ref, o_ref):
    r2 = par_ref[0]
    cxv = cx_ref[:]
    cyv = cy_ref[:]
    czv = cz_ref[:]
    for j in range(10):
        sl = slice(j * 1024, (j + 1) * 1024)
        dx = cxv - px_ref[:, sl]
        dy = cyv - py_ref[:, sl]
        dz = czv - pz_ref[:, sl]
        d2 = dx * dx + dy * dy + dz * dz
        hit = (d2 <= r2).astype(jnp.float32)
        w = jnp.dot(hit, pk_ref[:], preferred_element_type=jnp.float32)
        o_ref[:, j * 64:(j + 1) * 64] = w.astype(jnp.int32)


def _hitwords(cx, cy, cz, px1, py1, pz1, par, pack):
    return pl.pallas_call(
        _hm_body,
        grid=(20,),
        in_specs=[
            pl.BlockSpec((128, 1), lambda i: (i, 0)),
            pl.BlockSpec((128, 1), lambda i: (i, 0)),
            pl.BlockSpec((128, 1), lambda i: (i, 0)),
            pl.BlockSpec((1, _NP), lambda i: (0, 0)),
            pl.BlockSpec((1, _NP), lambda i: (0, 0)),
            pl.BlockSpec((1, _NP), lambda i: (0, 0)),
            pl.BlockSpec(memory_space=pltpu.SMEM),
            pl.BlockSpec((1024, 64), lambda i: (0, 0)),
        ],
        out_specs=pl.BlockSpec((128, _NG), lambda i: (i, 0)),
        out_shape=jax.ShapeDtypeStruct((_SP, _NG), jnp.int32),
    )(cx, cy, cz, px1, py1, pz1, par, pack)


def _sc_gather_max(mkflat, u):
    mesh = plsc.VectorSubcoreMesh(core_axis_name="c", subcore_axis_name="s")

    @functools.partial(
        pl.kernel,
        out_type=jax.ShapeDtypeStruct((_SP * _D,), jnp.float32),
        mesh=mesh,
        compiler_params=pltpu.CompilerParams(needs_layout_passes=False),
        scratch_types=[
            pltpu.VMEM((_PER * _NG,), jnp.int32),
            pltpu.VMEM((_PER,), jnp.int32),
            pltpu.VMEM((_PER,), jnp.int32),
            pltpu.VMEM((16, _D), jnp.float32),
            pltpu.VMEM((_PER * _D,), jnp.float32),
            pltpu.SemaphoreType.DMA,
        ],
    )
    def sc_kernel(mk_hbm, u_hbm, m_hbm,
                  mk_v, wlist_v, cand_v, rows_v, m_v, sem):
        wid = lax.axis_index("s") * 2 + lax.axis_index("c")
        base = wid * _PER

        pltpu.sync_copy(mk_hbm.at[pl.ds(base * _NG, _PER * _NG)], mk_v)
        lanes = lax.broadcasted_iota(jnp.int32, (16,), 0)

        def center(cl, carry):
            rowv = jnp.full((16,), cl, jnp.int32) + jnp.full(
                (16,), base, jnp.int32)
            for g in range(5):
                cand_v[pl.ds(g * 16, 16)] = rowv

            def scan_step(s, wcnt):
                wv = mk_v[pl.ds(cl * _NG + s * 16, 16)]
                nz = wv != 0

                def proc(wc):
                    cum = plsc.cumsum(jnp.where(nz, 1, 0).astype(jnp.int32))
                    off = wc + cum - 1
                    ok = nz & (off < _PER)
                    gv = jnp.full((16,), s * 16, jnp.int32) + lanes
                    plsc.store_scatter(wlist_v, [off], gv, mask=ok)
                    return wc + plsc.all_reduce_population_count(nz)

                return lax.cond(jnp.any(nz), proc, lambda wc: wc, wcnt)

            wcnt = lax.fori_loop(0, _NG // 16, scan_step,
                                 jnp.zeros((16,), jnp.int32))
            nw = jnp.minimum(lax.reduce_max(wcnt, axes=(0,)),
                             jnp.int32(_PER))

            def word_step(t, cnt):
                tv = jnp.full((16,), t, jnp.int32)
                gv = plsc.load_gather(wlist_v, [tv])
                wv = plsc.load_gather(
                    mk_v, [gv + jnp.full((16,), cl * _NG, jnp.int32)])
                hit = ((wv >> lanes) & 1) == 1
                cum = plsc.cumsum(jnp.where(hit, 1, 0).astype(jnp.int32))
                off = cnt + cum
                ok = hit & (off <= _K)
                jv = gv * 16 + lanes
                plsc.store_scatter(cand_v, [off], jv, mask=ok)
                return cnt + plsc.all_reduce_population_count(hit)

            cnt = lax.fori_loop(0, nw, word_step,
                                jnp.zeros((16,), jnp.int32))
            cnt_s = lax.reduce_max(cnt, axes=(0,))
            ne = 1 + jnp.minimum(cnt_s, jnp.int32(_K))
            nch = (ne + 15) // 16

            def gather_chunk(k, acc):
                pltpu.async_copy(
                    u_hbm.at[cand_v.at[pl.ds(k * 16, 16)]], rows_v, sem
                ).wait()
                new = []
                for dreg in range(8):
                    a = acc[dreg]
                    for rr in range(16):
                        a = jnp.maximum(a, rows_v[rr, pl.ds(dreg * 16, 16)])
                    new.append(a)
                return tuple(new)

            acc0 = tuple(
                jnp.full((16,), -jnp.inf, jnp.float32) for _ in range(8)
            )
            acc = lax.fori_loop(0, nch, gather_chunk, acc0)

            clv = jnp.full((16,), cl, jnp.int32)
            mbase = clv * _D + lanes
            for dreg in range(8):
                plsc.store_scatter(
                    m_v, [mbase + jnp.full((16,), dreg * 16, jnp.int32)],
                    acc[dreg])
            return carry

        lax.fori_loop(0, _PER, center, jnp.int32(0))
        pltpu.sync_copy(m_v, m_hbm.at[pl.ds(base * _D, _PER * _D)])

    return sc_kernel(mkflat, u)


def _tail_body(m_ref, pd_ref, wb_ref, b1_ref, w2_ref, b2_ref, o_ref):
    v = jnp.dot(pd_ref[:], wb_ref[:], preferred_element_type=jnp.float32)
    t = jnp.maximum(m_ref[:] - v + b1_ref[0:1, :], 0.0)
    y = jnp.dot(t, w2_ref[:], preferred_element_type=jnp.float32) + b2_ref[0:1, :]
    o_ref[:] = jnp.maximum(y, 0.0)


def _tail(m, pd8, wbp, b1, w2, b2):
    return pl.pallas_call(
        _tail_body,
        grid=(5,),
        in_specs=[
            pl.BlockSpec((512, _D), lambda i: (i, 0)),
            pl.BlockSpec((512, 8), lambda i: (i, 0)),
            pl.BlockSpec((8, _D), lambda i: (0, 0)),
            pl.BlockSpec((1, _D), lambda i: (0, 0)),
            pl.BlockSpec((_D, _D), lambda i: (0, 0)),
            pl.BlockSpec((1, _D), lambda i: (0, 0)),
        ],
        out_specs=pl.BlockSpec((512, _D), lambda i: (i, 0)),
        out_shape=jax.ShapeDtypeStruct((_SP, _D), jnp.float32),
    )(m, pd8, wbp, b1, w2, b2)


def kernel(x, pos, training, W1, b1, W2, b2):
    x = x.astype(jnp.float32)
    pos = pos.astype(jnp.float32)

    padn = _NP - _N
    px = jnp.concatenate([pos[:, 0], jnp.full((padn,), 1e9, jnp.float32)])
    py = jnp.concatenate([pos[:, 1], jnp.full((padn,), 1e9, jnp.float32)])
    pz = jnp.concatenate([pos[:, 2], jnp.full((padn,), 1e9, jnp.float32)])

    sel, pdx, pdy, pdz = _fps(
        px.reshape(80, 128), py.reshape(80, 128), pz.reshape(80, 128))

    pads = _SP - _S
    cpad = jnp.full((pads,), 2e9, jnp.float32)
    cx = jnp.concatenate([pdx, cpad]).reshape(_SP, 1)
    cy = jnp.concatenate([pdy, cpad]).reshape(_SP, 1)
    cz = jnp.concatenate([pdz, cpad]).reshape(_SP, 1)

    xp = jnp.pad(x, ((0, padn), (0, 0)))
    posp8 = jnp.pad(pos, ((0, padn), (0, 5)))
    wa = W1[:_D]
    wbp = jnp.pad(W1[_D:], ((0, 5), (0, 0)))
    u = _u_matmul(xp, posp8, wa, wbp)

    r2eff = jnp.where(training, jnp.float32(_R2), jnp.float32(-1.0))
    par = r2eff.reshape(1)

    p_ids = jnp.arange(1024, dtype=jnp.int32)
    w_ids = jnp.arange(64, dtype=jnp.int32)
    pack = jnp.where(
        (p_ids[:, None] // 16) == w_ids[None, :],
        jnp.exp2((p_ids % 16).astype(jnp.float32))[:, None],
        0.0,
    )

    mk = _hitwords(cx, cy, cz, px.reshape(1, _NP), py.reshape(1, _NP),
                   pz.reshape(1, _NP), par, pack)

    mflat = _sc_gather_max(mk.reshape(-1), u)
    m = mflat.reshape(_SP, _D)

    pd = jnp.stack([pdx, pdy, pdz], axis=1)
    pd8 = jnp.pad(pd, ((0, pads), (0, 5)))
    y = _tail(m, pd8, wbp, b1.reshape(1, _D), W2, b2.reshape(1, _D))
    return y[:_S], pd[:_S]


# speedup vs baseline: 1.1527x; 1.1527x over previous
import functools

import jax
import jax.numpy as jnp
from jax import lax
from jax.experimental import pallas as pl
from jax.experimental.pallas import tpu as pltpu
from jax.experimental.pallas import tpu_sc as plsc

_N = 10000
_D = 128
_S = 2500
_R2 = 0.3 * 0.3
_K = 64
_NP = 10240
_SP = 2560
_PER = 80
_NG = _NP // 16


def _fps_body(px_ref, py_ref, pz_ref, sel_ref, pdx_ref, pdy_ref, pdz_ref):
    px = px_ref[:]
    py = py_ref[:]
    pz = pz_ref[:]
    r = lax.broadcasted_iota(jnp.int32, (80, 128), 0)
    c = lax.broadcasted_iota(jnp.int32, (80, 128), 1)
    lin = r * 128 + c
    validm = lin < _N
    zero = jnp.float32(0.0)

    eq0 = lin == 0
    px0 = jnp.sum(jnp.where(eq0, px, zero), axis=(0, 1), keepdims=True)
    py0 = jnp.sum(jnp.where(eq0, py, zero), axis=(0, 1), keepdims=True)
    pz0 = jnp.sum(jnp.where(eq0, pz, zero), axis=(0, 1), keepdims=True)
    dx = px - px0
    dy = py - py0
    dz = pz - pz0
    mind = dx * dx + dy * dy + dz * dz
    mind = jnp.where(validm, mind, -jnp.inf)
    sel_ref[0:1, :] = jnp.zeros((1, 1), jnp.int32)
    pdx_ref[0:1, :] = px0
    pdy_ref[0:1, :] = py0
    pdz_ref[0:1, :] = pz0

    def body(i, mind):
        m = jnp.max(mind, axis=(0, 1), keepdims=True)
        nxtv = jnp.min(jnp.where(mind == m, lin, jnp.int32(2**30)),
                       axis=(0, 1), keepdims=True)
        eq = lin == nxtv
        pxv = jnp.sum(jnp.where(eq, px, zero), axis=(0, 1), keepdims=True)
        pyv = jnp.sum(jnp.where(eq, py, zero), axis=(0, 1), keepdims=True)
        pzv = jnp.sum(jnp.where(eq, pz, zero), axis=(0, 1), keepdims=True)
        ddx = px - pxv
        ddy = py - pyv
        ddz = pz - pzv
        d = ddx * ddx + ddy * ddy + ddz * ddz
        sel_ref[pl.ds(i, 1), :] = nxtv
        pdx_ref[pl.ds(i, 1), :] = pxv
        pdy_ref[pl.ds(i, 1), :] = pyv
        pdz_ref[pl.ds(i, 1), :] = pzv
        return jnp.minimum(mind, d)

    lax.fori_loop(1, _S, body, mind)


def _fps(px2d, py2d, pz2d):
    return pl.pallas_call(
        _fps_body,
        out_shape=[
            jax.ShapeDtypeStruct((_S, 1), jnp.int32),
            jax.ShapeDtypeStruct((_S, 1), jnp.float32),
            jax.ShapeDtypeStruct((_S, 1), jnp.float32),
            jax.ShapeDtypeStruct((_S, 1), jnp.float32),
        ],
    )(px2d, py2d, pz2d)


def _u_body(x_ref, p_ref, wa_ref, wb_ref, o_ref):
    o_ref[:] = jnp.dot(
        x_ref[:], wa_ref[:], preferred_element_type=jnp.float32
    ) + jnp.dot(p_ref[:], wb_ref[:], preferred_element_type=jnp.float32)


def _u_matmul(xp, posp8, wa, wbp):
    return pl.pallas_call(
        _u_body,
        grid=(20,),
        in_specs=[
            pl.BlockSpec((512, _D), lambda i: (i, 0)),
            pl.BlockSpec((512, 8), lambda i: (i, 0)),
            pl.BlockSpec((_D, _D), lambda i: (0, 0)),
            pl.BlockSpec((8, _D), lambda i: (0, 0)),
        ],
        out_specs=pl.BlockSpec((512, _D), lambda i: (i, 0)),
        out_shape=jax.ShapeDtypeStruct((_NP, _D), jnp.float32),
    )(xp, posp8, wa, wbp)


def _hm_body(cx_ref, cy_ref, cz_ref, px_ref, py_ref, pz_ref, par_ref,
             pk_ref, o_ref):
    r2 = par_ref[0]
    cxv = cx_ref[:]
    cyv = cy_ref[:]
    czv = cz_ref[:]
    for j in range(10):
        sl = slice(j * 1024, (j + 1) * 1024)
        dx = cxv - px_ref[:, sl]
        dy = cyv - py_ref[:, sl]
        dz = czv - pz_ref[:, sl]
        d2 = dx * dx + dy * dy + dz * dz
        hit = (d2 <= r2).astype(jnp.float32)
        w = jnp.dot(hit, pk_ref[:], preferred_element_type=jnp.float32)
        o_ref[:, j * 64:(j + 1) * 64] = w.astype(jnp.int32)


def _hitwords(cx, cy, cz, px1, py1, pz1, par, pack):
    return pl.pallas_call(
        _hm_body,
        grid=(20,),
        in_specs=[
            pl.BlockSpec((128, 1), lambda i: (i, 0)),
            pl.BlockSpec((128, 1), lambda i: (i, 0)),
            pl.BlockSpec((128, 1), lambda i: (i, 0)),
            pl.BlockSpec((1, _NP), lambda i: (0, 0)),
            pl.BlockSpec((1, _NP), lambda i: (0, 0)),
            pl.BlockSpec((1, _NP), lambda i: (0, 0)),
            pl.BlockSpec(memory_space=pltpu.SMEM),
            pl.BlockSpec((1024, 64), lambda i: (0, 0)),
        ],
        out_specs=pl.BlockSpec((128, _NG), lambda i: (i, 0)),
        out_shape=jax.ShapeDtypeStruct((_SP, _NG), jnp.int32),
    )(cx, cy, cz, px1, py1, pz1, par, pack)


def _sc_gather_max(mkflat, u):
    mesh = plsc.VectorSubcoreMesh(core_axis_name="c", subcore_axis_name="s")

    @functools.partial(
        pl.kernel,
        out_type=jax.ShapeDtypeStruct((_SP * _D,), jnp.float32),
        mesh=mesh,
        compiler_params=pltpu.CompilerParams(needs_layout_passes=False),
        scratch_types=[
            pltpu.VMEM((_PER * _NG,), jnp.int32),
            pltpu.VMEM((_PER,), jnp.int32),
            pltpu.VMEM((_PER,), jnp.int32),
            pltpu.VMEM((16, _D), jnp.float32),
            pltpu.VMEM((_PER * _D,), jnp.float32),
            pltpu.SemaphoreType.DMA,
        ],
    )
    def sc_kernel(mk_hbm, u_hbm, m_hbm,
                  mk_v, wlist_v, cand_v, rows_v, m_v, sem):
        wid = lax.axis_index("s") * 2 + lax.axis_index("c")
        base = wid * _PER

        pltpu.sync_copy(mk_hbm.at[pl.ds(base * _NG, _PER * _NG)], mk_v)
        lanes = lax.broadcasted_iota(jnp.int32, (16,), 0)

        def center(cl, carry):
            rowv = jnp.full((16,), cl, jnp.int32) + jnp.full(
                (16,), base, jnp.int32)
            for g in range(5):
                cand_v[pl.ds(g * 16, 16)] = rowv

            def scan_step(s, wcnt):
                wv = mk_v[pl.ds(cl * _NG + s * 16, 16)]
                nz = wv != 0

                def proc(wc):
                    cum = plsc.cumsum(jnp.where(nz, 1, 0).astype(jnp.int32))
                    off = wc + cum - 1
                    ok = nz & (off < _PER)
                    gv = jnp.full((16,), s * 16, jnp.int32) + lanes
                    plsc.store_scatter(wlist_v, [off], gv, mask=ok)
                    return wc + plsc.all_reduce_population_count(nz)

                return lax.cond(jnp.any(nz), proc, lambda wc: wc, wcnt)

            wcnt = lax.fori_loop(0, _NG // 16, scan_step,
                                 jnp.zeros((16,), jnp.int32))
            nw = jnp.minimum(lax.reduce_max(wcnt, axes=(0,)),
                             jnp.int32(_PER))

            def word_step(t, cnt):
                tv = jnp.full((16,), t, jnp.int32)
                gv = plsc.load_gather(wlist_v, [tv])
                wv = plsc.load_gather(
                    mk_v, [gv + jnp.full((16,), cl * _NG, jnp.int32)])
                hit = ((wv >> lanes) & 1) == 1
                cum = plsc.cumsum(jnp.where(hit, 1, 0).astype(jnp.int32))
                off = cnt + cum
                ok = hit & (off <= _K)
                jv = gv * 16 + lanes
                plsc.store_scatter(cand_v, [off], jv, mask=ok)
                return cnt + plsc.all_reduce_population_count(hit)

            cnt = lax.fori_loop(0, nw, word_step,
                                jnp.zeros((16,), jnp.int32))
            cnt_s = lax.reduce_max(cnt, axes=(0,))
            ne = 1 + jnp.minimum(cnt_s, jnp.int32(_K))
            nch = (ne + 15) // 16

            def gather_chunk(k, acc):
                pltpu.async_copy(
                    u_hbm.at[cand_v.at[pl.ds(k * 16, 16)]], rows_v, sem
                ).wait()
                new = []
                for dreg in range(8):
                    a = acc[dreg]
                    for rr in range(16):
                        a = jnp.maximum(a, rows_v[rr, pl.ds(dreg * 16, 16)])
                    new.append(a)
                return tuple(new)

            acc0 = tuple(
                jnp.full((16,), -jnp.inf, jnp.float32) for _ in range(8)
            )
            acc = lax.fori_loop(0, nch, gather_chunk, acc0)

            clv = jnp.full((16,), cl, jnp.int32)
            mbase = clv * _D + lanes
            for dreg in range(8):
                plsc.store_scatter(
                    m_v, [mbase + jnp.full((16,), dreg * 16, jnp.int32)],
                    acc[dreg])
            return carry

        lax.fori_loop(0, _PER, center, jnp.int32(0))
        pltpu.sync_copy(m_v, m_hbm.at[pl.ds(base * _D, _PER * _D)])

    return sc_kernel(mkflat, u)


def _tail_body(m_ref, pd_ref, wb_ref, b1_ref, w2_ref, b2_ref, o_ref):
    v = jnp.dot(pd_ref[:], wb_ref[:], preferred_element_type=jnp.float32)
    t = jnp.maximum(m_ref[:] - v + b1_ref[0:1, :], 0.0)
    y = jnp.dot(t, w2_ref[:], preferred_element_type=jnp.float32) + b2_ref[0:1, :]
    o_ref[:] = jnp.maximum(y, 0.0)


def _tail(m, pd8, wbp, b1, w2, b2):
    return pl.pallas_call(
        _tail_body,
        grid=(5,),
        in_specs=[
            pl.BlockSpec((512, _D), lambda i: (i, 0)),
            pl.BlockSpec((512, 8), lambda i: (i, 0)),
            pl.BlockSpec((8, _D), lambda i: (0, 0)),
            pl.BlockSpec((1, _D), lambda i: (0, 0)),
            pl.BlockSpec((_D, _D), lambda i: (0, 0)),
            pl.BlockSpec((1, _D), lambda i: (0, 0)),
        ],
        out_specs=pl.BlockSpec((512, _D), lambda i: (i, 0)),
        out_shape=jax.ShapeDtypeStruct((_SP, _D), jnp.float32),
    )(m, pd8, wbp, b1, w2, b2)


def kernel(x, pos, training, W1, b1, W2, b2):
    x = x.astype(jnp.float32)
    pos = pos.astype(jnp.float32)

    padn = _NP - _N
    px = jnp.concatenate([pos[:, 0], jnp.full((padn,), 1e9, jnp.float32)])
    py = jnp.concatenate([pos[:, 1], jnp.full((padn,), 1e9, jnp.float32)])
    pz = jnp.concatenate([pos[:, 2], jnp.full((padn,), 1e9, jnp.float32)])

    sel, pdx, pdy, pdz = _fps(
        px.reshape(80, 128), py.reshape(80, 128), pz.reshape(80, 128))
    sel = sel.reshape(_S)
    pdx = pdx.reshape(_S)
    pdy = pdy.reshape(_S)
    pdz = pdz.reshape(_S)

    pads = _SP - _S
    cpad = jnp.full((pads,), 2e9, jnp.float32)
    cx = jnp.concatenate([pdx, cpad]).reshape(_SP, 1)
    cy = jnp.concatenate([pdy, cpad]).reshape(_SP, 1)
    cz = jnp.concatenate([pdz, cpad]).reshape(_SP, 1)

    xp = jnp.pad(x, ((0, padn), (0, 0)))
    posp8 = jnp.pad(pos, ((0, padn), (0, 5)))
    wa = W1[:_D]
    wbp = jnp.pad(W1[_D:], ((0, 5), (0, 0)))
    u = _u_matmul(xp, posp8, wa, wbp)

    r2eff = jnp.where(training, jnp.float32(_R2), jnp.float32(-1.0))
    par = r2eff.reshape(1)

    p_ids = jnp.arange(1024, dtype=jnp.int32)
    w_ids = jnp.arange(64, dtype=jnp.int32)
    pack = jnp.where(
        (p_ids[:, None] // 16) == w_ids[None, :],
        jnp.exp2((p_ids % 16).astype(jnp.float32))[:, None],
        0.0,
    )

    mk = _hitwords(cx, cy, cz, px.reshape(1, _NP), py.reshape(1, _NP),
                   pz.reshape(1, _NP), par, pack)

    mflat = _sc_gather_max(mk.reshape(-1), u)
    m = mflat.reshape(_SP, _D)

    pd = jnp.stack([pdx, pdy, pdz], axis=1)
    pd8 = jnp.pad(pd, ((0, pads), (0, 5)))
    y = _tail(m, pd8, wbp, b1.reshape(1, _D), W2, b2.reshape(1, _D))
    return y[:_S], pd[:_S]


# speedup vs baseline: 1.4202x; 1.2321x over previous
import functools

import jax
import jax.numpy as jnp
from jax import lax
from jax.experimental import pallas as pl
from jax.experimental.pallas import tpu as pltpu
from jax.experimental.pallas import tpu_sc as plsc

_N = 10000
_D = 128
_S = 2500
_R2 = 0.3 * 0.3
_K = 64
_NP = 10240
_SP = 2560
_PER = 80
_NG = _NP // 16


def _fps_body(px_ref, py_ref, pz_ref, sel_ref, pdx_ref, pdy_ref, pdz_ref):
    px = px_ref[:]
    py = py_ref[:]
    pz = pz_ref[:]
    r = lax.broadcasted_iota(jnp.int32, (80, 128), 0)
    c = lax.broadcasted_iota(jnp.int32, (80, 128), 1)
    lin = r * 128 + c
    validm = lin < _N
    zero = jnp.float32(0.0)

    eq0 = lin == 0
    px0 = jnp.sum(jnp.where(eq0, px, zero), axis=(0, 1), keepdims=True)
    py0 = jnp.sum(jnp.where(eq0, py, zero), axis=(0, 1), keepdims=True)
    pz0 = jnp.sum(jnp.where(eq0, pz, zero), axis=(0, 1), keepdims=True)
    dx = px - px0
    dy = py - py0
    dz = pz - pz0
    mind = dx * dx + dy * dy + dz * dz
    mind = jnp.where(validm, mind, -jnp.inf)
    sel_ref[0:1, :] = jnp.zeros((1, 1), jnp.int32)
    pdx_ref[0:1, :] = px0
    pdy_ref[0:1, :] = py0
    pdz_ref[0:1, :] = pz0

    big = jnp.int32(2**30)

    def body(i, mind):
        colmax = jnp.max(mind, axis=0, keepdims=True)
        m = jnp.max(colmax, axis=1, keepdims=True)
        eqm = mind == m
        a0 = jnp.min(jnp.where(eqm, lin, big), axis=0, keepdims=True)
        fm = lin == a0
        ax = jnp.sum(jnp.where(fm, px, zero), axis=0, keepdims=True)
        ay = jnp.sum(jnp.where(fm, py, zero), axis=0, keepdims=True)
        az = jnp.sum(jnp.where(fm, pz, zero), axis=0, keepdims=True)
        nxtv = jnp.min(a0, axis=1, keepdims=True)
        lm = a0 == nxtv
        packed = jnp.where(lm, jnp.concatenate([ax, ay, az], axis=0), zero)
        red3 = jnp.sum(packed, axis=1, keepdims=True)
        pxv = red3[0:1, :]
        pyv = red3[1:2, :]
        pzv = red3[2:3, :]
        ddx = px - pxv
        ddy = py - pyv
        ddz = pz - pzv
        d = ddx * ddx + ddy * ddy + ddz * ddz
        sel_ref[pl.ds(i, 1), :] = nxtv
        pdx_ref[pl.ds(i, 1), :] = pxv
        pdy_ref[pl.ds(i, 1), :] = pyv
        pdz_ref[pl.ds(i, 1), :] = pzv
        return jnp.minimum(mind, d)

    lax.fori_loop(1, _S, body, mind)


def _fps(px2d, py2d, pz2d):
    return pl.pallas_call(
        _fps_body,
        out_shape=[
            jax.ShapeDtypeStruct((_S, 1), jnp.int32),
            jax.ShapeDtypeStruct((_S, 1), jnp.float32),
            jax.ShapeDtypeStruct((_S, 1), jnp.float32),
            jax.ShapeDtypeStruct((_S, 1), jnp.float32),
        ],
    )(px2d, py2d, pz2d)


def _u_body(x_ref, p_ref, wa_ref, wb_ref, o_ref):
    o_ref[:] = jnp.dot(
        x_ref[:], wa_ref[:], preferred_element_type=jnp.float32
    ) + jnp.dot(p_ref[:], wb_ref[:], preferred_element_type=jnp.float32)


def _u_matmul(xp, posp8, wa, wbp):
    return pl.pallas_call(
        _u_body,
        grid=(20,),
        in_specs=[
            pl.BlockSpec((512, _D), lambda i: (i, 0)),
            pl.BlockSpec((512, 8), lambda i: (i, 0)),
            pl.BlockSpec((_D, _D), lambda i: (0, 0)),
            pl.BlockSpec((8, _D), lambda i: (0, 0)),
        ],
        out_specs=pl.BlockSpec((512, _D), lambda i: (i, 0)),
        out_shape=jax.ShapeDtypeStruct((_NP, _D), jnp.float32),
    )(xp, posp8, wa, wbp)


def _hm_body(cx_ref, cy_ref, cz_ref, px_ref, py_ref, pz_ref, par_ref,
             pk_ref, o_ref):
    r2 = par_ref[0]
    cxv = cx_ref[:]
    cyv = cy_ref[:]
    czv = cz_ref[:]
    for j in range(10):
        sl = slice(j * 1024, (j + 1) * 1024)
        dx = cxv - px_ref[:, sl]
        dy = cyv - py_ref[:, sl]
        dz = czv - pz_ref[:, sl]
        d2 = dx * dx + dy * dy + dz * dz
        hit = (d2 <= r2).astype(jnp.float32)
        w = jnp.dot(hit, pk_ref[:], preferred_element_type=jnp.float32)
        o_ref[:, j * 64:(j + 1) * 64] = w.astype(jnp.int32)


def _hitwords(cx, cy, cz, px1, py1, pz1, par, pack):
    return pl.pallas_call(
        _hm_body,
        grid=(20,),
        in_specs=[
            pl.BlockSpec((128, 1), lambda i: (i, 0)),
            pl.BlockSpec((128, 1), lambda i: (i, 0)),
            pl.BlockSpec((128, 1), lambda i: (i, 0)),
            pl.BlockSpec((1, _NP), lambda i: (0, 0)),
            pl.BlockSpec((1, _NP), lambda i: (0, 0)),
            pl.BlockSpec((1, _NP), lambda i: (0, 0)),
            pl.BlockSpec(memory_space=pltpu.SMEM),
            pl.BlockSpec((1024, 64), lambda i: (0, 0)),
        ],
        out_specs=pl.BlockSpec((128, _NG), lambda i: (i, 0)),
        out_shape=jax.ShapeDtypeStruct((_SP, _NG), jnp.int32),
    )(cx, cy, cz, px1, py1, pz1, par, pack)


def _sc_gather_max(mkflat, u):
    mesh = plsc.VectorSubcoreMesh(core_axis_name="c", subcore_axis_name="s")

    @functools.partial(
        pl.kernel,
        out_type=jax.ShapeDtypeStruct((_SP * _D,), jnp.float32),
        mesh=mesh,
        compiler_params=pltpu.CompilerParams(needs_layout_passes=False),
        scratch_types=[
            pltpu.VMEM((_PER * _NG,), jnp.int32),
            pltpu.VMEM((_PER,), jnp.int32),
            pltpu.VMEM((_PER,), jnp.int32),
            pltpu.VMEM((16, _D), jnp.float32),
            pltpu.VMEM((_PER * _D,), jnp.float32),
            pltpu.SemaphoreType.DMA,
        ],
    )
    def sc_kernel(mk_hbm, u_hbm, m_hbm,
                  mk_v, wlist_v, cand_v, rows_v, m_v, sem):
        wid = lax.axis_index("s") * 2 + lax.axis_index("c")
        base = wid * _PER

        pltpu.sync_copy(mk_hbm.at[pl.ds(base * _NG, _PER * _NG)], mk_v)
        lanes = lax.broadcasted_iota(jnp.int32, (16,), 0)

        def center(cl, carry):
            rowv = jnp.full((16,), cl, jnp.int32) + jnp.full(
                (16,), base, jnp.int32)
            for g in range(5):
                cand_v[pl.ds(g * 16, 16)] = rowv

            def scan_step(s, wcnt):
                wv = mk_v[pl.ds(cl * _NG + s * 16, 16)]
                nz = wv != 0

                def proc(wc):
                    cum = plsc.cumsum(jnp.where(nz, 1, 0).astype(jnp.int32))
                    off = wc + cum - 1
                    ok = nz & (off < _PER)
                    gv = jnp.full((16,), s * 16, jnp.int32) + lanes
                    plsc.store_scatter(wlist_v, [off], gv, mask=ok)
                    return wc + plsc.all_reduce_population_count(nz)

                return lax.cond(jnp.any(nz), proc, lambda wc: wc, wcnt)

            wcnt = lax.fori_loop(0, _NG // 16, scan_step,
                                 jnp.zeros((16,), jnp.int32))
            nw = jnp.minimum(lax.reduce_max(wcnt, axes=(0,)),
                             jnp.int32(_PER))

            def word_step(t, cnt):
                tv = jnp.full((16,), t, jnp.int32)
                gv = plsc.load_gather(wlist_v, [tv])
                wv = plsc.load_gather(
                    mk_v, [gv + jnp.full((16,), cl * _NG, jnp.int32)])
                hit = ((wv >> lanes) & 1) == 1
                cum = plsc.cumsum(jnp.where(hit, 1, 0).astype(jnp.int32))
                off = cnt + cum
                ok = hit & (off <= _K)
                jv = gv * 16 + lanes
                plsc.store_scatter(cand_v, [off], jv, mask=ok)
                return cnt + plsc.all_reduce_population_count(hit)

            cnt = lax.fori_loop(0, nw, word_step,
                                jnp.zeros((16,), jnp.int32))
            cnt_s = lax.reduce_max(cnt, axes=(0,))
            ne = 1 + jnp.minimum(cnt_s, jnp.int32(_K))
            nch = (ne + 15) // 16

            def gather_chunk(k, acc):
                pltpu.async_copy(
                    u_hbm.at[cand_v.at[pl.ds(k * 16, 16)]], rows_v, sem
                ).wait()
                new = []
                for dreg in range(8):
                    a = acc[dreg]
                    for rr in range(16):
                        a = jnp.maximum(a, rows_v[rr, pl.ds(dreg * 16, 16)])
                    new.append(a)
                return tuple(new)

            acc0 = tuple(
                jnp.full((16,), -jnp.inf, jnp.float32) for _ in range(8)
            )
            acc = lax.fori_loop(0, nch, gather_chunk, acc0)

            clv = jnp.full((16,), cl, jnp.int32)
            mbase = clv * _D + lanes
            for dreg in range(8):
                plsc.store_scatter(
                    m_v, [mbase + jnp.full((16,), dreg * 16, jnp.int32)],
                    acc[dreg])
            return carry

        lax.fori_loop(0, _PER, center, jnp.int32(0))
        pltpu.sync_copy(m_v, m_hbm.at[pl.ds(base * _D, _PER * _D)])

    return sc_kernel(mkflat, u)


def _tail_body(m_ref, pd_ref, wb_ref, b1_ref, w2_ref, b2_ref, o_ref):
    v = jnp.dot(pd_ref[:], wb_ref[:], preferred_element_type=jnp.float32)
    t = jnp.maximum(m_ref[:] - v + b1_ref[0:1, :], 0.0)
    y = jnp.dot(t, w2_ref[:], preferred_element_type=jnp.float32) + b2_ref[0:1, :]
    o_ref[:] = jnp.maximum(y, 0.0)


def _tail(m, pd8, wbp, b1, w2, b2):
    return pl.pallas_call(
        _tail_body,
        grid=(5,),
        in_specs=[
            pl.BlockSpec((512, _D), lambda i: (i, 0)),
            pl.BlockSpec((512, 8), lambda i: (i, 0)),
            pl.BlockSpec((8, _D), lambda i: (0, 0)),
            pl.BlockSpec((1, _D), lambda i: (0, 0)),
            pl.BlockSpec((_D, _D), lambda i: (0, 0)),
            pl.BlockSpec((1, _D), lambda i: (0, 0)),
        ],
        out_specs=pl.BlockSpec((512, _D), lambda i: (i, 0)),
        out_shape=jax.ShapeDtypeStruct((_SP, _D), jnp.float32),
    )(m, pd8, wbp, b1, w2, b2)


def kernel(x, pos, training, W1, b1, W2, b2):
    x = x.astype(jnp.float32)
    pos = pos.astype(jnp.float32)

    padn = _NP - _N
    px = jnp.concatenate([pos[:, 0], jnp.full((padn,), 1e9, jnp.float32)])
    py = jnp.concatenate([pos[:, 1], jnp.full((padn,), 1e9, jnp.float32)])
    pz = jnp.concatenate([pos[:, 2], jnp.full((padn,), 1e9, jnp.float32)])

    sel, pdx, pdy, pdz = _fps(
        px.reshape(80, 128), py.reshape(80, 128), pz.reshape(80, 128))
    sel = sel.reshape(_S)
    pdx = pdx.reshape(_S)
    pdy = pdy.reshape(_S)
    pdz = pdz.reshape(_S)

    pads = _SP - _S
    cpad = jnp.full((pads,), 2e9, jnp.float32)
    cx = jnp.concatenate([pdx, cpad]).reshape(_SP, 1)
    cy = jnp.concatenate([pdy, cpad]).reshape(_SP, 1)
    cz = jnp.concatenate([pdz, cpad]).reshape(_SP, 1)

    xp = jnp.pad(x, ((0, padn), (0, 0)))
    posp8 = jnp.pad(pos, ((0, padn), (0, 5)))
    wa = W1[:_D]
    wbp = jnp.pad(W1[_D:], ((0, 5), (0, 0)))
    u = _u_matmul(xp, posp8, wa, wbp)

    r2eff = jnp.where(training, jnp.float32(_R2), jnp.float32(-1.0))
    par = r2eff.reshape(1)

    p_ids = jnp.arange(1024, dtype=jnp.int32)
    w_ids = jnp.arange(64, dtype=jnp.int32)
    pack = jnp.where(
        (p_ids[:, None] // 16) == w_ids[None, :],
        jnp.exp2((p_ids % 16).astype(jnp.float32))[:, None],
        0.0,
    )

    mk = _hitwords(cx, cy, cz, px.reshape(1, _NP), py.reshape(1, _NP),
                   pz.reshape(1, _NP), par, pack)

    mflat = _sc_gather_max(mk.reshape(-1), u)
    m = mflat.reshape(_SP, _D)

    pd = jnp.stack([pdx, pdy, pdz], axis=1)
    pd8 = jnp.pad(pd, ((0, pads), (0, 5)))
    y = _tail(m, pd8, wbp, b1.reshape(1, _D), W2, b2.reshape(1, _D))
    return y[:_S], pd[:_S]


# speedup vs baseline: 1.4581x; 1.0267x over previous
import functools

import jax
import jax.numpy as jnp
from jax import lax
from jax.experimental import pallas as pl
from jax.experimental.pallas import tpu as pltpu
from jax.experimental.pallas import tpu_sc as plsc

_N = 10000
_D = 128
_S = 2500
_R2 = 0.3 * 0.3
_K = 64
_NP = 10240
_SP = 2560
_PER = 80
_NG = _NP // 16


def _fps_body(px_ref, py_ref, pz_ref, sel_ref, pdx_ref, pdy_ref, pdz_ref):
    px = px_ref[:]
    py = py_ref[:]
    pz = pz_ref[:]
    r = lax.broadcasted_iota(jnp.int32, (80, 128), 0)
    c = lax.broadcasted_iota(jnp.int32, (80, 128), 1)
    lin = r * 128 + c
    validm = lin < _N
    zero = jnp.float32(0.0)

    eq0 = lin == 0
    px0 = jnp.sum(jnp.where(eq0, px, zero), axis=(0, 1), keepdims=True)
    py0 = jnp.sum(jnp.where(eq0, py, zero), axis=(0, 1), keepdims=True)
    pz0 = jnp.sum(jnp.where(eq0, pz, zero), axis=(0, 1), keepdims=True)
    dx = px - px0
    dy = py - py0
    dz = pz - pz0
    mind = dx * dx + dy * dy + dz * dz
    mind = jnp.where(validm, mind, -jnp.inf)
    sel_ref[0:1, :] = jnp.zeros((1, 1), jnp.int32)
    pdx_ref[0:1, :] = px0
    pdy_ref[0:1, :] = py0
    pdz_ref[0:1, :] = pz0

    big = jnp.int32(2**30)

    def body(i, mind):
        colmax = jnp.max(mind, axis=0, keepdims=True)
        m = jnp.max(colmax, axis=1, keepdims=True)
        eqm = mind == m
        a0 = jnp.min(jnp.where(eqm, lin, big), axis=0, keepdims=True)
        fm = lin == a0
        ax = jnp.sum(jnp.where(fm, px, zero), axis=0, keepdims=True)
        ay = jnp.sum(jnp.where(fm, py, zero), axis=0, keepdims=True)
        az = jnp.sum(jnp.where(fm, pz, zero), axis=0, keepdims=True)
        nxtv = jnp.min(a0, axis=1, keepdims=True)
        lm = a0 == nxtv
        packed = jnp.where(lm, jnp.concatenate([ax, ay, az], axis=0), zero)
        red3 = jnp.sum(packed, axis=1, keepdims=True)
        pxv = red3[0:1, :]
        pyv = red3[1:2, :]
        pzv = red3[2:3, :]
        ddx = px - pxv
        ddy = py - pyv
        ddz = pz - pzv
        d = ddx * ddx + ddy * ddy + ddz * ddz
        sel_ref[pl.ds(i, 1), :] = nxtv
        pdx_ref[pl.ds(i, 1), :] = pxv
        pdy_ref[pl.ds(i, 1), :] = pyv
        pdz_ref[pl.ds(i, 1), :] = pzv
        return jnp.minimum(mind, d)

    lax.fori_loop(1, _S, body, mind)


def _fps(px2d, py2d, pz2d):
    return pl.pallas_call(
        _fps_body,
        out_shape=[
            jax.ShapeDtypeStruct((_S, 1), jnp.int32),
            jax.ShapeDtypeStruct((_S, 1), jnp.float32),
            jax.ShapeDtypeStruct((_S, 1), jnp.float32),
            jax.ShapeDtypeStruct((_S, 1), jnp.float32),
        ],
    )(px2d, py2d, pz2d)


def _u_body(x_ref, p_ref, wa_ref, wb_ref, o_ref):
    o_ref[:] = jnp.dot(
        x_ref[:], wa_ref[:], preferred_element_type=jnp.float32
    ) + jnp.dot(p_ref[:], wb_ref[:], preferred_element_type=jnp.float32)


def _u_matmul(xp, posp8, wa, wbp):
    return pl.pallas_call(
        _u_body,
        grid=(20,),
        in_specs=[
            pl.BlockSpec((512, _D), lambda i: (i, 0)),
            pl.BlockSpec((512, 8), lambda i: (i, 0)),
            pl.BlockSpec((_D, _D), lambda i: (0, 0)),
            pl.BlockSpec((8, _D), lambda i: (0, 0)),
        ],
        out_specs=pl.BlockSpec((512, _D), lambda i: (i, 0)),
        out_shape=jax.ShapeDtypeStruct((_NP, _D), jnp.float32),
    )(xp, posp8, wa, wbp)


def _hm_body(cx_ref, cy_ref, cz_ref, px_ref, py_ref, pz_ref, par_ref,
             pk_ref, o_ref):
    r2 = par_ref[0]
    cxv = cx_ref[:]
    cyv = cy_ref[:]
    czv = cz_ref[:]
    for j in range(10):
        sl = slice(j * 1024, (j + 1) * 1024)
        dx = cxv - px_ref[:, sl]
        dy = cyv - py_ref[:, sl]
        dz = czv - pz_ref[:, sl]
        d2 = dx * dx + dy * dy + dz * dz
        hit = (d2 <= r2).astype(jnp.float32)
        w = jnp.dot(hit, pk_ref[:], preferred_element_type=jnp.float32)
        o_ref[:, j * 64:(j + 1) * 64] = w.astype(jnp.int32)


def _hitwords(cx, cy, cz, px1, py1, pz1, par, pack):
    return pl.pallas_call(
        _hm_body,
        grid=(20,),
        in_specs=[
            pl.BlockSpec((128, 1), lambda i: (i, 0)),
            pl.BlockSpec((128, 1), lambda i: (i, 0)),
            pl.BlockSpec((128, 1), lambda i: (i, 0)),
            pl.BlockSpec((1, _NP), lambda i: (0, 0)),
            pl.BlockSpec((1, _NP), lambda i: (0, 0)),
            pl.BlockSpec((1, _NP), lambda i: (0, 0)),
            pl.BlockSpec(memory_space=pltpu.SMEM),
            pl.BlockSpec((1024, 64), lambda i: (0, 0)),
        ],
        out_specs=pl.BlockSpec((128, _NG), lambda i: (i, 0)),
        out_shape=jax.ShapeDtypeStruct((_SP, _NG), jnp.int32),
    )(cx, cy, cz, px1, py1, pz1, par, pack)


def _sc_gather_max(mkflat, u):
    mesh = plsc.VectorSubcoreMesh(core_axis_name="c", subcore_axis_name="s")

    @functools.partial(
        pl.kernel,
        out_type=jax.ShapeDtypeStruct((_SP * _D,), jnp.float32),
        mesh=mesh,
        compiler_params=pltpu.CompilerParams(needs_layout_passes=False),
        scratch_types=[
            pltpu.VMEM((_PER * _NG,), jnp.int32),
            pltpu.VMEM((_PER,), jnp.int32),
            pltpu.VMEM((_PER,), jnp.int32),
            pltpu.VMEM((_PER, _D), jnp.float32),
            pltpu.VMEM((_PER * _D,), jnp.float32),
            pltpu.SemaphoreType.DMA,
        ],
    )
    def sc_kernel(mk_hbm, u_hbm, m_hbm,
                  mk_v, wlist_v, cand_v, rows_v, m_v, sem):
        wid = lax.axis_index("s") * 2 + lax.axis_index("c")
        base = wid * _PER

        pltpu.sync_copy(mk_hbm.at[pl.ds(base * _NG, _PER * _NG)], mk_v)
        lanes = lax.broadcasted_iota(jnp.int32, (16,), 0)

        def center(cl, carry):
            rowv = jnp.full((16,), cl, jnp.int32) + jnp.full(
                (16,), base, jnp.int32)
            for g in range(5):
                cand_v[pl.ds(g * 16, 16)] = rowv

            def scan_step(s, wcnt):
                wv = mk_v[pl.ds(cl * _NG + s * 16, 16)]
                nz = wv != 0

                def proc(wc):
                    cum = plsc.cumsum(jnp.where(nz, 1, 0).astype(jnp.int32))
                    off = wc + cum - 1
                    ok = nz & (off < _PER)
                    gv = jnp.full((16,), s * 16, jnp.int32) + lanes
                    plsc.store_scatter(wlist_v, [off], gv, mask=ok)
                    return wc + plsc.all_reduce_population_count(nz)

                return lax.cond(jnp.any(nz), proc, lambda wc: wc, wcnt)

            wcnt = lax.fori_loop(0, _NG // 16, scan_step,
                                 jnp.zeros((16,), jnp.int32))
            nw = jnp.minimum(lax.reduce_max(wcnt, axes=(0,)),
                             jnp.int32(_PER))

            def word_step(t, cnt):
                tv = jnp.full((16,), t, jnp.int32)
                gv = plsc.load_gather(wlist_v, [tv])
                wv = plsc.load_gather(
                    mk_v, [gv + jnp.full((16,), cl * _NG, jnp.int32)])
                hit = ((wv >> lanes) & 1) == 1
                cum = plsc.cumsum(jnp.where(hit, 1, 0).astype(jnp.int32))
                off = cnt + cum
                ok = hit & (off <= _K)
                jv = gv * 16 + lanes
                plsc.store_scatter(cand_v, [off], jv, mask=ok)
                return cnt + plsc.all_reduce_population_count(hit)

            cnt = lax.fori_loop(0, nw, word_step,
                                jnp.zeros((16,), jnp.int32))
            cnt_s = lax.reduce_max(cnt, axes=(0,))
            ne = 1 + jnp.minimum(cnt_s, jnp.int32(_K))
            nch = (ne + 15) // 16

            def fire_chunk(k, carry):
                pltpu.async_copy(
                    u_hbm.at[cand_v.at[pl.ds(k * 16, 16)]],
                    rows_v.at[pl.ds(k * 16, 16), :], sem)
                return carry

            lax.fori_loop(0, nch, fire_chunk, jnp.int32(0))

            def drain_chunk(k, carry):
                pltpu.make_async_copy(
                    u_hbm.at[cand_v.at[pl.ds(k * 16, 16)]],
                    rows_v.at[pl.ds(k * 16, 16), :], sem).wait()
                return carry

            lax.fori_loop(0, nch, drain_chunk, jnp.int32(0))

            def max_chunk(k, acc):
                new = []
                for dreg in range(8):
                    a = acc[dreg]
                    for rr in range(16):
                        a = jnp.maximum(
                            a, rows_v[k * 16 + rr, pl.ds(dreg * 16, 16)])
                    new.append(a)
                return tuple(new)

            acc0 = tuple(
                jnp.full((16,), -jnp.inf, jnp.float32) for _ in range(8)
            )
            acc = lax.fori_loop(0, nch, max_chunk, acc0)

            clv = jnp.full((16,), cl, jnp.int32)
            mbase = clv * _D + lanes
            for dreg in range(8):
                plsc.store_scatter(
                    m_v, [mbase + jnp.full((16,), dreg * 16, jnp.int32)],
                    acc[dreg])
            return carry

        lax.fori_loop(0, _PER, center, jnp.int32(0))
        pltpu.sync_copy(m_v, m_hbm.at[pl.ds(base * _D, _PER * _D)])

    return sc_kernel(mkflat, u)


def _tail_body(m_ref, pd_ref, wb_ref, b1_ref, w2_ref, b2_ref, o_ref):
    v = jnp.dot(pd_ref[:], wb_ref[:], preferred_element_type=jnp.float32)
    t = jnp.maximum(m_ref[:] - v + b1_ref[0:1, :], 0.0)
    y = jnp.dot(t, w2_ref[:], preferred_element_type=jnp.float32) + b2_ref[0:1, :]
    o_ref[:] = jnp.maximum(y, 0.0)


def _tail(m, pd8, wbp, b1, w2, b2):
    return pl.pallas_call(
        _tail_body,
        grid=(5,),
        in_specs=[
            pl.BlockSpec((512, _D), lambda i: (i, 0)),
            pl.BlockSpec((512, 8), lambda i: (i, 0)),
            pl.BlockSpec((8, _D), lambda i: (0, 0)),
            pl.BlockSpec((1, _D), lambda i: (0, 0)),
            pl.BlockSpec((_D, _D), lambda i: (0, 0)),
            pl.BlockSpec((1, _D), lambda i: (0, 0)),
        ],
        out_specs=pl.BlockSpec((512, _D), lambda i: (i, 0)),
        out_shape=jax.ShapeDtypeStruct((_SP, _D), jnp.float32),
    )(m, pd8, wbp, b1, w2, b2)


def kernel(x, pos, training, W1, b1, W2, b2):
    x = x.astype(jnp.float32)
    pos = pos.astype(jnp.float32)

    padn = _NP - _N
    px = jnp.concatenate([pos[:, 0], jnp.full((padn,), 1e9, jnp.float32)])
    py = jnp.concatenate([pos[:, 1], jnp.full((padn,), 1e9, jnp.float32)])
    pz = jnp.concatenate([pos[:, 2], jnp.full((padn,), 1e9, jnp.float32)])

    sel, pdx, pdy, pdz = _fps(
        px.reshape(80, 128), py.reshape(80, 128), pz.reshape(80, 128))
    sel = sel.reshape(_S)
    pdx = pdx.reshape(_S)
    pdy = pdy.reshape(_S)
    pdz = pdz.reshape(_S)

    pads = _SP - _S
    cpad = jnp.full((pads,), 2e9, jnp.float32)
    cx = jnp.concatenate([pdx, cpad]).reshape(_SP, 1)
    cy = jnp.concatenate([pdy, cpad]).reshape(_SP, 1)
    cz = jnp.concatenate([pdz, cpad]).reshape(_SP, 1)

    xp = jnp.pad(x, ((0, padn), (0, 0)))
    posp8 = jnp.pad(pos, ((0, padn), (0, 5)))
    wa = W1[:_D]
    wbp = jnp.pad(W1[_D:], ((0, 5), (0, 0)))
    u = _u_matmul(xp, posp8, wa, wbp)

    r2eff = jnp.where(training, jnp.float32(_R2), jnp.float32(-1.0))
    par = r2eff.reshape(1)

    p_ids = jnp.arange(1024, dtype=jnp.int32)
    w_ids = jnp.arange(64, dtype=jnp.int32)
    pack = jnp.where(
        (p_ids[:, None] // 16) == w_ids[None, :],
        jnp.exp2((p_ids % 16).astype(jnp.float32))[:, None],
        0.0,
    )

    mk = _hitwords(cx, cy, cz, px.reshape(1, _NP), py.reshape(1, _NP),
                   pz.reshape(1, _NP), par, pack)

    mflat = _sc_gather_max(mk.reshape(-1), u)
    m = mflat.reshape(_SP, _D)

    pd = jnp.stack([pdx, pdy, pdz], axis=1)
    pd8 = jnp.pad(pd, ((0, pads), (0, 5)))
    y = _tail(m, pd8, wbp, b1.reshape(1, _D), W2, b2.reshape(1, _D))
    return y[:_S], pd[:_S]
